# Initial kernel scaffold; baseline (speedup 1.0000x reference)
import jax
import jax.numpy as jnp
from jax.experimental import pallas as pl


def kernel(x, codebook):
    raise NotImplementedError("write your pallas kernel here")


# speedup vs baseline: 9.3649x; 9.3649x over previous
import functools

import jax
import jax.numpy as jnp
from jax.experimental import pallas as pl
from jax.experimental.pallas import tpu as pltpu

K = 1024
D = 256
H = 8
BLK = 256
TAU = 1.0
COMMIT = 0.25
ENT_RATIO = 0.1
ENT_TEMP = 0.01


def _vq_block(x_ref, cb_ref, q_ref, loss_ref, ti_ref, tw_ref, enc_ref,
              avgp_acc, sacc, *, n_tokens, n_blocks):
    i = pl.program_id(0)

    @pl.when(i == 0)
    def _init():
        avgp_acc[...] = jnp.zeros_like(avgp_acc)
        sacc[0] = 0.0
        sacc[1] = 0.0

    x = x_ref[...]
    cb = cb_ref[...]

    ab = jax.lax.dot_general(x, cb, (((1,), (1,)), ((), ())),
                             preferred_element_type=jnp.float32)
    x2 = jnp.sum(x * x, axis=1, keepdims=True)
    b2 = jnp.sum(cb * cb, axis=1)[None, :]
    d = x2 - 2.0 * ab + b2

    a = d * (-1.0 / ENT_TEMP)
    m = jnp.max(a, axis=1, keepdims=True)
    e = jnp.exp(a - m)
    z = jnp.sum(e, axis=1, keepdims=True)
    p = e / z
    s_ent = jnp.log(z[:, 0]) - jnp.sum(e * (a - m), axis=1) / z[:, 0]
    avgp_acc[...] += jnp.sum(p, axis=0, keepdims=True)
    sacc[0] += jnp.sum(s_ent)

    iota_k = jax.lax.broadcasted_iota(jnp.int32, (BLK, K), 1)
    dwork = d
    idxs = []
    dists = []
    for _ in range(H):
        mval = jnp.min(dwork, axis=1, keepdims=True)
        hit = dwork == mval
        idx = jnp.min(jnp.where(hit, iota_k, K), axis=1, keepdims=True)
        idxs.append(idx)
        dists.append(mval)
        dwork = jnp.where(iota_k == idx, jnp.inf, dwork)
    top_i = jnp.concatenate(idxs, axis=1)
    top_d = jnp.concatenate(dists, axis=1)

    w = jnp.exp((top_d[:, :1] - top_d) * (1.0 / TAU))
    tw = w / jnp.sum(w, axis=1, keepdims=True)
    ti_ref[...] = top_i
    tw_ref[...] = tw

    enc = jnp.zeros((BLK, K), jnp.float32)
    for h in range(H):
        enc += jnp.where(iota_k == top_i[:, h:h + 1], tw[:, h:h + 1], 0.0)
    enc_ref[...] = enc

    q = jax.lax.dot_general(enc, cb, (((1,), (0,)), ((), ())),
                            preferred_element_type=jnp.float32)
    q_ref[...] = q
    r = q - x
    sacc[1] += jnp.sum(r * r)

    @pl.when(i == n_blocks - 1)
    def _fin():
        navg = 1.0 / n_tokens
        avg_p = avgp_acc[...] * navg
        avg_ent = -jnp.sum(avg_p * jnp.log(avg_p + 1e-5))
        ent_loss = ENT_RATIO * (sacc[0] * navg - avg_ent)
        mse = sacc[1] * (navg / D)
        loss_ref[...] = jnp.reshape((1.0 + COMMIT) * mse + ent_loss, (1, 1))


@jax.jit
def _vq(x2d, cb):
    n_tokens = x2d.shape[0]
    n_blocks = n_tokens // BLK
    grid = (n_blocks,)
    kern = functools.partial(_vq_block, n_tokens=n_tokens, n_blocks=n_blocks)
    return pl.pallas_call(
        kern,
        grid=grid,
        in_specs=[
            pl.BlockSpec((BLK, D), lambda i: (i, 0)),
            pl.BlockSpec((K, D), lambda i: (0, 0)),
        ],
        out_specs=[
            pl.BlockSpec((BLK, D), lambda i: (i, 0)),
            pl.BlockSpec((1, 1), lambda i: (0, 0)),
            pl.BlockSpec((BLK, H), lambda i: (i, 0)),
            pl.BlockSpec((BLK, H), lambda i: (i, 0)),
            pl.BlockSpec((BLK, K), lambda i: (i, 0)),
        ],
        out_shape=[
            jax.ShapeDtypeStruct((n_tokens, D), jnp.float32),
            jax.ShapeDtypeStruct((1, 1), jnp.float32),
            jax.ShapeDtypeStruct((n_tokens, H), jnp.int32),
            jax.ShapeDtypeStruct((n_tokens, H), jnp.float32),
            jax.ShapeDtypeStruct((n_tokens, K), jnp.float32),
        ],
        scratch_shapes=[
            pltpu.VMEM((1, K), jnp.float32),
            pltpu.SMEM((2,), jnp.float32),
        ],
    )(x2d, cb)


def kernel(x, codebook):
    b, t, d = x.shape
    x2d = x.reshape(b * t, d)
    q, loss, ti, tw, enc = _vq(x2d, codebook)
    return (q.reshape(b, t, d), loss[0, 0], ti.reshape(b, t, H),
            tw.reshape(b, t, H), enc.reshape(b, t, K))
